# baseline (speedup 1.0000x reference)
import functools

import jax
import jax.numpy as jnp
from jax.experimental import pallas as pl
from jax.experimental.pallas import tpu as pltpu


def _normalize(x):
    norm = jnp.sqrt(jnp.sum(x * x, axis=-1, keepdims=True))
    return x / jnp.maximum(norm, 1e-12)


def _threshold_kernel(x_ref, t_ref):
    x = x_ref[0]
    xn = _normalize(x)
    m = x.shape[0]
    adj = jax.lax.dot_general(xn, xn, (((1,), (1,)), ((), ())),
                              preferred_element_type=jnp.float32)
    row = jax.lax.broadcasted_iota(jnp.int32, adj.shape, 0)
    col = jax.lax.broadcasted_iota(jnp.int32, adj.shape, 1)
    adj = jnp.where(row == col, 0.0, adj)
    m1 = jnp.max(adj, axis=0, keepdims=True)
    first = jnp.min(jnp.where(adj == m1, row, m), axis=0, keepdims=True)
    masked = jnp.where(row == first, -3.0, adj)
    t_ref[0] = jnp.max(masked, axis=0, keepdims=True)


def _output_kernel(rows, x_ref, t_ref, o_ref):
    i = pl.program_id(1)
    x = x_ref[0]
    xn = _normalize(x)
    xr = _normalize(x_ref[0, pl.ds(i * rows, rows), :])
    m = x.shape[0]
    adj = jax.lax.dot_general(xr, xn, (((1,), (1,)), ((), ())),
                              preferred_element_type=jnp.float32)
    row = jax.lax.broadcasted_iota(jnp.int32, adj.shape, 0) + i * rows
    col = jax.lax.broadcasted_iota(jnp.int32, adj.shape, 1)
    adj = jnp.where(row == col, 0.0, adj)
    m1 = jnp.max(adj, axis=1, keepdims=True)
    first = jnp.min(jnp.where(adj == m1, col, m), axis=1, keepdims=True)
    m2 = jnp.max(jnp.where(col == first, -3.0, adj), axis=1, keepdims=True)
    ind = (adj >= m2).astype(jnp.float32) + (adj >= t_ref[0]).astype(jnp.float32)
    o_ref[0] = adj * ind * 0.5


def kernel(x, W1, b1, W2, b2):
    b, m, d = x.shape
    rows = 512

    t = pl.pallas_call(
        _threshold_kernel,
        grid=(b,),
        in_specs=[pl.BlockSpec((1, m, d), lambda i: (i, 0, 0))],
        out_specs=pl.BlockSpec((1, 1, m), lambda i: (i, 0, 0)),
        out_shape=jax.ShapeDtypeStruct((b, 1, m), jnp.float32),
        compiler_params=pltpu.CompilerParams(
            dimension_semantics=("parallel",)),
    )(x)

    out = pl.pallas_call(
        functools.partial(_output_kernel, rows),
        grid=(b, m // rows),
        in_specs=[
            pl.BlockSpec((1, m, d), lambda i, j: (i, 0, 0)),
            pl.BlockSpec((1, 1, m), lambda i, j: (i, 0, 0)),
        ],
        out_specs=pl.BlockSpec((1, rows, m), lambda i, j: (i, j, 0)),
        out_shape=jax.ShapeDtypeStruct((b, m, m), jnp.float32),
        compiler_params=pltpu.CompilerParams(
            dimension_semantics=("parallel", "parallel")),
    )(x, t)
    return out


# speedup vs baseline: 1.2761x; 1.2761x over previous
import functools

import jax
import jax.numpy as jnp
from jax.experimental import pallas as pl
from jax.experimental.pallas import tpu as pltpu


def _normalize(x):
    norm = jnp.sqrt(jnp.sum(x * x, axis=-1, keepdims=True))
    return x / jnp.maximum(norm, 1e-12)


def _threshold_kernel(x_ref, t_ref):
    x = x_ref[0]
    xn = _normalize(x)
    m = x.shape[0]
    adj = jax.lax.dot_general(xn, xn, (((1,), (1,)), ((), ())),
                              preferred_element_type=jnp.float32)
    row = jax.lax.broadcasted_iota(jnp.int32, adj.shape, 0)
    col = jax.lax.broadcasted_iota(jnp.int32, adj.shape, 1)
    adj = jnp.where(row == col, 0.0, adj)
    m1 = jnp.max(adj, axis=0, keepdims=True)
    first = jnp.min(jnp.where(adj == m1, row, m), axis=0, keepdims=True)
    masked = jnp.where(row == first, -3.0, adj)
    t_ref[0] = jnp.max(masked, axis=0, keepdims=True)


def _output_kernel(rows, x_ref, t_ref, o_ref):
    i = pl.program_id(1)
    x = x_ref[0]
    xn = _normalize(x)
    xr = _normalize(x_ref[0, pl.ds(i * rows, rows), :])
    adj = jax.lax.dot_general(xr, xn, (((1,), (1,)), ((), ())),
                              preferred_element_type=jnp.float32)
    row = jax.lax.broadcasted_iota(jnp.int32, adj.shape, 0) + i * rows
    col = jax.lax.broadcasted_iota(jnp.int32, adj.shape, 1)
    adj = jnp.where(row == col, 0.0, adj)
    t_col = t_ref[0]
    t_row = jnp.swapaxes(t_ref[0, :, pl.ds(i * rows, rows)], 0, 1)
    w = jnp.where(adj >= t_row, 0.5, 0.0) + jnp.where(adj >= t_col, 0.5, 0.0)
    o_ref[0] = adj * w


def kernel(x, W1, b1, W2, b2):
    b, m, d = x.shape
    rows = 512

    t = pl.pallas_call(
        _threshold_kernel,
        grid=(b,),
        in_specs=[pl.BlockSpec((1, m, d), lambda i: (i, 0, 0))],
        out_specs=pl.BlockSpec((1, 1, m), lambda i: (i, 0, 0)),
        out_shape=jax.ShapeDtypeStruct((b, 1, m), jnp.float32),
        compiler_params=pltpu.CompilerParams(
            dimension_semantics=("parallel",)),
    )(x)

    out = pl.pallas_call(
        functools.partial(_output_kernel, rows),
        grid=(b, m // rows),
        in_specs=[
            pl.BlockSpec((1, m, d), lambda i, j: (i, 0, 0)),
            pl.BlockSpec((1, 1, m), lambda i, j: (i, 0, 0)),
        ],
        out_specs=pl.BlockSpec((1, rows, m), lambda i, j: (i, j, 0)),
        out_shape=jax.ShapeDtypeStruct((b, m, m), jnp.float32),
        compiler_params=pltpu.CompilerParams(
            dimension_semantics=("parallel", "parallel")),
    )(x, t)
    return out
